# Initial kernel scaffold; baseline (speedup 1.0000x reference)
import jax
import jax.numpy as jnp
from jax.experimental import pallas as pl


def kernel(hidden_states, edge_index_i, edge_index_ii, edge_index_iii, edge_index_a, W_i, W_ii, W_iii, W_a, ln_gamma, ln_beta, ff_w1, ff_b1, ff_w2, ff_b2):
    raise NotImplementedError("write your pallas kernel here")


# speedup vs baseline: 1.4483x; 1.4483x over previous
import functools

import jax
import jax.numpy as jnp
from jax import lax
from jax.experimental import pallas as pl
from jax.experimental.pallas import tpu as pltpu
from jax.experimental.pallas import tpu_sc as plsc

HIDDEN = 256
INTER = 1024
N_NODES = 10000
N_EDGES = 160000
EPS = 1e-06

_Q = HIDDEN // 4
_N_SLABS = 16
_N_TILES = 16
_EDGE_CH = 80
_EPT = N_EDGES // _N_TILES
_CH_PER_TILE = _EPT // _EDGE_CH
_RPT = 624
_REM_BASE = _RPT * _N_TILES
_REM = N_NODES - _REM_BASE


_LN_BLK = 2000


def _ln_body(x_ref, gam_ref, bet_ref, h_ref, q0_ref, q1_ref, q2_ref, q3_ref):
    x = x_ref[...]
    mu = jnp.mean(x, axis=-1, keepdims=True)
    xc = x - mu
    var = jnp.mean(xc * xc, axis=-1, keepdims=True)
    h = xc * lax.rsqrt(var + EPS) * gam_ref[...] + bet_ref[...]
    h_ref[...] = h
    q0_ref[...] = h[:, 0 * _Q:1 * _Q]
    q1_ref[...] = h[:, 1 * _Q:2 * _Q]
    q2_ref[...] = h[:, 2 * _Q:3 * _Q]
    q3_ref[...] = h[:, 3 * _Q:4 * _Q]


_ln_call = pl.pallas_call(
    _ln_body,
    grid=(N_NODES // _LN_BLK,),
    in_specs=[
        pl.BlockSpec((_LN_BLK, HIDDEN), lambda i: (i, 0)),
        pl.BlockSpec((1, HIDDEN), lambda i: (0, 0)),
        pl.BlockSpec((1, HIDDEN), lambda i: (0, 0)),
    ],
    out_specs=[pl.BlockSpec((_LN_BLK, HIDDEN), lambda i: (i, 0))] +
              [pl.BlockSpec((_LN_BLK, _Q), lambda i: (i, 0))] * 4,
    out_shape=[jax.ShapeDtypeStruct((N_NODES, HIDDEN), jnp.float32)] +
              [jax.ShapeDtypeStruct((N_NODES, _Q), jnp.float32)] * 4,
)


_sc_mesh = plsc.VectorSubcoreMesh(core_axis_name="c", subcore_axis_name="s")


@functools.partial(
    pl.kernel,
    mesh=_sc_mesh,
    out_type=jax.ShapeDtypeStruct((_N_SLABS, N_NODES, _Q), jnp.float32),
    scratch_types=[
        pltpu.VMEM((_EDGE_CH,), jnp.int32),
        pltpu.VMEM((_EDGE_CH,), jnp.int32),
        pltpu.VMEM((_EDGE_CH, _Q), jnp.float32),
        pltpu.VMEM((_RPT, _Q), jnp.float32),
        pltpu.VMEM_SHARED((N_NODES, _Q), jnp.float32),
        pltpu.SemaphoreType.DMA,
    ],
    compiler_params=pltpu.CompilerParams(use_tc_tiling_on_sc=False),
)
def _sc_scatter(q0, q1, q2, q3, src0, dst0, src1, dst1, src2, dst2,
                src3, dst3, zeros_hbm, out_hbm,
                src_v, dst_v, rows_v, zrows_v, accum, sem):
    cid = lax.axis_index("c")
    sid = lax.axis_index("s")
    pltpu.sync_copy(zeros_hbm, zrows_v)
    quarters = (q0, q1, q2, q3)
    edges = ((src0, dst0), (src1, dst1), (src2, dst2), (src3, dst3))
    for s in range(_N_SLABS):
        k, q = divmod(s, 4)

        @pl.when(cid == s // 8)
        def _slab(k=k, q=q, s=s):
            pltpu.sync_copy(zrows_v, accum.at[pl.ds(sid * _RPT, _RPT)])

            @pl.when(sid == 0)
            def _zrem():
                pltpu.sync_copy(zrows_v.at[pl.ds(0, _REM)],
                                accum.at[pl.ds(_REM_BASE, _REM)])

            plsc.subcore_barrier()

            def body(j, carry):
                off = sid * _EPT + j * _EDGE_CH
                pltpu.sync_copy(edges[k][0].at[pl.ds(off, _EDGE_CH)], src_v)
                pltpu.sync_copy(edges[k][1].at[pl.ds(off, _EDGE_CH)], dst_v)
                pltpu.async_copy(quarters[q].at[src_v], rows_v, sem).wait()
                pltpu.sync_copy(rows_v, accum.at[dst_v], add=True)
                return carry

            lax.fori_loop(0, _CH_PER_TILE, body, 0)
            plsc.subcore_barrier()
            pltpu.sync_copy(accum.at[pl.ds(sid * _RPT, _RPT)],
                            out_hbm.at[s, pl.ds(sid * _RPT, _RPT)])

            @pl.when(sid == 0)
            def _wrem():
                pltpu.sync_copy(accum.at[pl.ds(_REM_BASE, _REM)],
                                out_hbm.at[s, pl.ds(_REM_BASE, _REM)])


_DN_BLK = 1000


def _dense_body(h_ref, g_ref, wc_ref, w1_ref, b1_ref, w2_ref, b2_ref, o_ref):
    acc = jnp.zeros((_DN_BLK, HIDDEN), jnp.float32)
    for s in range(_N_SLABS):
        acc += jnp.dot(g_ref[s], wc_ref[s], preferred_element_type=jnp.float32)
    h2 = h_ref[...] + acc
    inter = jnp.dot(h2, w1_ref[...], preferred_element_type=jnp.float32)
    inter = inter + b1_ref[...]
    inter = jnp.where(inter >= 0, inter, 0.01 * inter)
    ff = jnp.dot(inter, w2_ref[...], preferred_element_type=jnp.float32)
    o_ref[...] = h2 + ff + b2_ref[...]


_dense_call = pl.pallas_call(
    _dense_body,
    grid=(N_NODES // _DN_BLK,),
    in_specs=[
        pl.BlockSpec((_DN_BLK, HIDDEN), lambda i: (i, 0)),
        pl.BlockSpec((_N_SLABS, _DN_BLK, _Q), lambda i: (0, i, 0)),
        pl.BlockSpec((_N_SLABS, _Q, HIDDEN), lambda i: (0, 0, 0)),
        pl.BlockSpec((HIDDEN, INTER), lambda i: (0, 0)),
        pl.BlockSpec((1, INTER), lambda i: (0, 0)),
        pl.BlockSpec((INTER, HIDDEN), lambda i: (0, 0)),
        pl.BlockSpec((1, HIDDEN), lambda i: (0, 0)),
    ],
    out_specs=pl.BlockSpec((_DN_BLK, HIDDEN), lambda i: (i, 0)),
    out_shape=jax.ShapeDtypeStruct((N_NODES, HIDDEN), jnp.float32),
)


def kernel(hidden_states, edge_index_i, edge_index_ii, edge_index_iii,
           edge_index_a, W_i, W_ii, W_iii, W_a, ln_gamma, ln_beta,
           ff_w1, ff_b1, ff_w2, ff_b2):
    h, q0, q1, q2, q3 = _ln_call(hidden_states,
                                 ln_gamma.reshape(1, HIDDEN),
                                 ln_beta.reshape(1, HIDDEN))
    er = []
    for e in (edge_index_i, edge_index_ii, edge_index_iii, edge_index_a):
        e32 = e.astype(jnp.int32)
        er += [e32[0], e32[1]]
    zeros = jnp.zeros((_RPT, _Q), jnp.float32)
    g = _sc_scatter(q0, q1, q2, q3, *er, zeros)
    wc = jnp.stack([W[i * _Q:(i + 1) * _Q]
                    for W in (W_i, W_ii, W_iii, W_a)
                    for i in range(4)])
    return _dense_call(h, g, wc,
                       ff_w1, ff_b1.reshape(1, INTER),
                       ff_w2, ff_b2.reshape(1, HIDDEN))


# speedup vs baseline: 2.7551x; 1.9024x over previous
import functools

import jax
import jax.numpy as jnp
from jax import lax
from jax.experimental import pallas as pl
from jax.experimental.pallas import tpu as pltpu
from jax.experimental.pallas import tpu_sc as plsc

HIDDEN = 256
INTER = 1024
N_NODES = 10000
N_EDGES = 160000
EPS = 1e-06

_Q = HIDDEN // 4
_N_SLABS = 16
_N_TILES = 16
_EDGE_CH = 80
_EPT = N_EDGES // _N_TILES
_CH_PER_TILE = _EPT // _EDGE_CH
_RPT = 624
_REM_BASE = _RPT * _N_TILES
_REM = N_NODES - _REM_BASE


_LN_BLK = 2000


def _ln_body(x_ref, gam_ref, bet_ref, h_ref, q0_ref, q1_ref, q2_ref, q3_ref):
    x = x_ref[...]
    mu = jnp.mean(x, axis=-1, keepdims=True)
    xc = x - mu
    var = jnp.mean(xc * xc, axis=-1, keepdims=True)
    h = xc * lax.rsqrt(var + EPS) * gam_ref[...] + bet_ref[...]
    h_ref[...] = h
    q0_ref[...] = h[:, 0 * _Q:1 * _Q]
    q1_ref[...] = h[:, 1 * _Q:2 * _Q]
    q2_ref[...] = h[:, 2 * _Q:3 * _Q]
    q3_ref[...] = h[:, 3 * _Q:4 * _Q]


_ln_call = pl.pallas_call(
    _ln_body,
    grid=(N_NODES // _LN_BLK,),
    in_specs=[
        pl.BlockSpec((_LN_BLK, HIDDEN), lambda i: (i, 0)),
        pl.BlockSpec((1, HIDDEN), lambda i: (0, 0)),
        pl.BlockSpec((1, HIDDEN), lambda i: (0, 0)),
    ],
    out_specs=[pl.BlockSpec((_LN_BLK, HIDDEN), lambda i: (i, 0))] +
              [pl.BlockSpec((_LN_BLK, _Q), lambda i: (i, 0))] * 4,
    out_shape=[jax.ShapeDtypeStruct((N_NODES, HIDDEN), jnp.float32)] +
              [jax.ShapeDtypeStruct((N_NODES, _Q), jnp.float32)] * 4,
)


_sc_mesh = plsc.VectorSubcoreMesh(core_axis_name="c", subcore_axis_name="s")


_IBLK = 25
_NBLK = _CH_PER_TILE // _IBLK


@functools.partial(
    pl.kernel,
    mesh=_sc_mesh,
    out_type=jax.ShapeDtypeStruct((_N_SLABS, N_NODES, _Q), jnp.float32),
    scratch_types=[
        pltpu.VMEM((2, _IBLK, _EDGE_CH), jnp.int32),
        pltpu.VMEM((2, _IBLK, _EDGE_CH), jnp.int32),
        pltpu.VMEM((2, _EDGE_CH, _Q), jnp.float32),
        pltpu.VMEM((_RPT, _Q), jnp.float32),
        pltpu.VMEM_SHARED((N_NODES, _Q), jnp.float32),
        pltpu.SemaphoreType.DMA,
    ],
    compiler_params=pltpu.CompilerParams(use_tc_tiling_on_sc=False),
)
def _sc_scatter(q0, q1, q2, q3, src0, dst0, src1, dst1, src2, dst2,
                src3, dst3, zeros_hbm, out_hbm,
                sblk, dblk, rows, zrows_v, accum, sem):
    cid = lax.axis_index("c")
    sid = lax.axis_index("s")
    pltpu.sync_copy(zeros_hbm, zrows_v)
    quarters = (q0, q1, q2, q3)
    edges = ((src0, dst0), (src1, dst1), (src2, dst2), (src3, dst3))
    for s in range(_N_SLABS):
        k, q = divmod(s, 4)

        @pl.when(cid == s // 8)
        def _slab(k=k, q=q, s=s):
            src2d, dst2d = edges[k]
            hq = quarters[q]
            pltpu.sync_copy(zrows_v, accum.at[pl.ds(sid * _RPT, _RPT)])

            @pl.when(sid == 0)
            def _zrem():
                pltpu.sync_copy(zrows_v.at[pl.ds(0, _REM)],
                                accum.at[pl.ds(_REM_BASE, _REM)])

            plsc.subcore_barrier()

            row0 = sid * _CH_PER_TILE
            pltpu.sync_copy(src2d.at[pl.ds(row0, _IBLK)], sblk.at[0])
            pltpu.sync_copy(dst2d.at[pl.ds(row0, _IBLK)], dblk.at[0])
            pltpu.async_copy(hq.at[sblk.at[0, 0]], rows.at[0], sem)

            def body(j, carry):
                b = lax.rem(j, 2)
                blk = lax.div(j, _IBLK)
                p = lax.rem(blk, 2)
                jj = lax.rem(j, _IBLK)
                pltpu.make_async_copy(hq.at[sblk.at[p, jj]],
                                      rows.at[b], sem).wait()

                @pl.when((jj == _IBLK - 1) & (j < _CH_PER_TILE - 1))
                def _fetch():
                    nb = blk + 1
                    nrow = row0 + nb * _IBLK
                    pltpu.sync_copy(src2d.at[pl.ds(nrow, _IBLK)],
                                    sblk.at[1 - p])
                    pltpu.sync_copy(dst2d.at[pl.ds(nrow, _IBLK)],
                                    dblk.at[1 - p])

                @pl.when(j < _CH_PER_TILE - 1)
                def _next():
                    j1 = j + 1
                    p1 = lax.rem(lax.div(j1, _IBLK), 2)
                    jj1 = lax.rem(j1, _IBLK)
                    pltpu.async_copy(hq.at[sblk.at[p1, jj1]],
                                     rows.at[1 - b], sem)

                pltpu.sync_copy(rows.at[b], accum.at[dblk.at[p, jj]],
                                add=True)
                return carry

            lax.fori_loop(0, _CH_PER_TILE, body, 0)
            plsc.subcore_barrier()
            pltpu.sync_copy(accum.at[pl.ds(sid * _RPT, _RPT)],
                            out_hbm.at[s, pl.ds(sid * _RPT, _RPT)])

            @pl.when(sid == 0)
            def _wrem():
                pltpu.sync_copy(accum.at[pl.ds(_REM_BASE, _REM)],
                                out_hbm.at[s, pl.ds(_REM_BASE, _REM)])


_DN_BLK = 1000


def _dense_body(h_ref, g_ref, wc_ref, w1_ref, b1_ref, w2_ref, b2_ref, o_ref):
    acc = jnp.zeros((_DN_BLK, HIDDEN), jnp.float32)
    for s in range(_N_SLABS):
        acc += jnp.dot(g_ref[s], wc_ref[s], preferred_element_type=jnp.float32)
    h2 = h_ref[...] + acc
    inter = jnp.dot(h2, w1_ref[...], preferred_element_type=jnp.float32)
    inter = inter + b1_ref[...]
    inter = jnp.where(inter >= 0, inter, 0.01 * inter)
    ff = jnp.dot(inter, w2_ref[...], preferred_element_type=jnp.float32)
    o_ref[...] = h2 + ff + b2_ref[...]


_dense_call = pl.pallas_call(
    _dense_body,
    grid=(N_NODES // _DN_BLK,),
    in_specs=[
        pl.BlockSpec((_DN_BLK, HIDDEN), lambda i: (i, 0)),
        pl.BlockSpec((_N_SLABS, _DN_BLK, _Q), lambda i: (0, i, 0)),
        pl.BlockSpec((_N_SLABS, _Q, HIDDEN), lambda i: (0, 0, 0)),
        pl.BlockSpec((HIDDEN, INTER), lambda i: (0, 0)),
        pl.BlockSpec((1, INTER), lambda i: (0, 0)),
        pl.BlockSpec((INTER, HIDDEN), lambda i: (0, 0)),
        pl.BlockSpec((1, HIDDEN), lambda i: (0, 0)),
    ],
    out_specs=pl.BlockSpec((_DN_BLK, HIDDEN), lambda i: (i, 0)),
    out_shape=jax.ShapeDtypeStruct((N_NODES, HIDDEN), jnp.float32),
)


def kernel(hidden_states, edge_index_i, edge_index_ii, edge_index_iii,
           edge_index_a, W_i, W_ii, W_iii, W_a, ln_gamma, ln_beta,
           ff_w1, ff_b1, ff_w2, ff_b2):
    h, q0, q1, q2, q3 = _ln_call(hidden_states,
                                 ln_gamma.reshape(1, HIDDEN),
                                 ln_beta.reshape(1, HIDDEN))
    er = []
    for e in (edge_index_i, edge_index_ii, edge_index_iii, edge_index_a):
        e32 = e.astype(jnp.int32)
        er += [e32[0].reshape(-1, _EDGE_CH), e32[1].reshape(-1, _EDGE_CH)]
    zeros = jnp.zeros((_RPT, _Q), jnp.float32)
    g = _sc_scatter(q0, q1, q2, q3, *er, zeros)
    wc = jnp.stack([W[i * _Q:(i + 1) * _Q]
                    for W in (W_i, W_ii, W_iii, W_a)
                    for i in range(4)])
    return _dense_call(h, g, wc,
                       ff_w1, ff_b1.reshape(1, INTER),
                       ff_w2, ff_b2.reshape(1, HIDDEN))
